# baseline (speedup 1.0000x reference)
import jax
import jax.numpy as jnp
from jax.experimental import pallas as pl
from jax.experimental.pallas import tpu as pltpu

_B = 8
_C_OUT = 64
_STRIDE = 12


def _dma_kernel(x_ref, o_ref, sem):
    copies = []
    for b in range(_B):
        for c in range(_C_OUT):
            copies.append(
                pltpu.make_async_copy(
                    x_ref.at[b, c * _STRIDE], o_ref.at[b, c], sem
                )
            )
    for cp in copies:
        cp.start()
    for cp in copies:
        cp.wait()


def kernel(x):
    return pl.pallas_call(
        _dma_kernel,
        in_specs=[pl.BlockSpec(memory_space=pl.ANY)],
        out_specs=pl.BlockSpec(memory_space=pl.ANY),
        out_shape=jax.ShapeDtypeStruct((_B, _C_OUT, 224, 224), x.dtype),
        scratch_shapes=[pltpu.SemaphoreType.DMA],
    )(x)


# speedup vs baseline: 1.0661x; 1.0661x over previous
import jax
import jax.numpy as jnp
from jax.experimental import pallas as pl
from jax.experimental.pallas import tpu as pltpu

_B = 8
_C_OUT = 64
_STRIDE = 12


def _dma_kernel(x_ref, o_ref, sem):
    copies = [
        pltpu.make_async_copy(x_ref.at[b, :, 0], o_ref.at[b], sem)
        for b in range(_B)
    ]
    for cp in copies:
        cp.start()
    for cp in copies:
        cp.wait()


def kernel(x):
    xv = x.reshape(_B, _C_OUT, _STRIDE, 224, 224)
    return pl.pallas_call(
        _dma_kernel,
        in_specs=[pl.BlockSpec(memory_space=pl.ANY)],
        out_specs=pl.BlockSpec(memory_space=pl.ANY),
        out_shape=jax.ShapeDtypeStruct((_B, _C_OUT, 224, 224), x.dtype),
        scratch_shapes=[pltpu.SemaphoreType.DMA],
    )(xv)


# speedup vs baseline: 3.6125x; 3.3884x over previous
import jax
import jax.numpy as jnp
from jax.experimental import pallas as pl
from jax.experimental.pallas import tpu as pltpu

_B = 8
_C_OUT = 64
_STRIDE = 12


def _copy_kernel(x_ref, o_ref):
    o_ref[...] = x_ref[...]


def kernel(x):
    return pl.pallas_call(
        _copy_kernel,
        grid=(_C_OUT,),
        in_specs=[
            pl.BlockSpec((_B, 1, 224, 224), lambda c: (0, c * _STRIDE, 0, 0)),
        ],
        out_specs=pl.BlockSpec((_B, 1, 224, 224), lambda c: (0, c, 0, 0)),
        out_shape=jax.ShapeDtypeStruct((_B, _C_OUT, 224, 224), x.dtype),
    )(x)


# speedup vs baseline: 3.6276x; 1.0042x over previous
import jax
import jax.numpy as jnp
from jax.experimental import pallas as pl
from jax.experimental.pallas import tpu as pltpu

_B = 8
_C_OUT = 64
_STRIDE = 12


def _gather_kernel(x_ref, o_ref, sem):
    b = pl.program_id(0)
    copies = [
        pltpu.make_async_copy(
            x_ref.at[b, c * _STRIDE], o_ref.at[0, c], sem
        )
        for c in range(_C_OUT)
    ]
    for cp in copies:
        cp.start()
    for cp in copies:
        cp.wait()


def kernel(x):
    return pl.pallas_call(
        _gather_kernel,
        grid=(_B,),
        in_specs=[pl.BlockSpec(memory_space=pl.ANY)],
        out_specs=pl.BlockSpec((1, _C_OUT, 224, 224), lambda b: (b, 0, 0, 0)),
        out_shape=jax.ShapeDtypeStruct((_B, _C_OUT, 224, 224), x.dtype),
        scratch_shapes=[pltpu.SemaphoreType.DMA],
    )(x)
